# baseline (speedup 1.0000x reference)
import functools

import jax
import jax.numpy as jnp
from jax.experimental import pallas as pl
from jax.experimental.pallas import tpu as pltpu

_CHUNK = 32768
_NBUF = 1


def _body(x_hbm, w_ref, b_ref, o_hbm, x_buf, o_buf, in_sems, out_sems):
    n = x_hbm.shape[0]
    num_chunks = n // _CHUNK
    w = w_ref[...]
    b = b_ref[...]

    def start_in(c, slot):
        pltpu.make_async_copy(
            x_hbm.at[pl.ds(c * _CHUNK, _CHUNK), :],
            x_buf.at[slot],
            in_sems.at[slot],
        ).start()

    for s in range(_NBUF):
        start_in(s, s)

    def step(c, _):
        slot = jax.lax.rem(c, _NBUF)
        pltpu.make_async_copy(
            x_hbm.at[pl.ds(c * _CHUNK, _CHUNK), :],
            x_buf.at[slot],
            in_sems.at[slot],
        ).wait()
        @pl.when(c >= _NBUF)
        def _():
            pltpu.make_async_copy(
                o_buf.at[slot],
                o_hbm.at[pl.ds((c - _NBUF) * _CHUNK, _CHUNK), :],
                out_sems.at[slot],
            ).wait()

        o_buf[slot] = (
            jnp.dot(x_buf[slot], w, preferred_element_type=jnp.float32) + b
        )
        pltpu.make_async_copy(
            o_buf.at[slot],
            o_hbm.at[pl.ds(c * _CHUNK, _CHUNK), :],
            out_sems.at[slot],
        ).start()
        @pl.when(c + _NBUF < num_chunks)
        def _():
            start_in(c + _NBUF, slot)

        return _

    jax.lax.fori_loop(0, num_chunks, step, None)

    for s in range(_NBUF):
        c = num_chunks - _NBUF + s
        slot = jax.lax.rem(jnp.int32(c), _NBUF)
        pltpu.make_async_copy(
            o_buf.at[slot],
            o_hbm.at[pl.ds(c * _CHUNK, _CHUNK), :],
            out_sems.at[slot],
        ).wait()


def kernel(input, W, b):
    n, in_f = input.shape
    out_f = W.shape[1]
    b2 = b.reshape(1, out_f)
    out = pl.pallas_call(
        _body,
        in_specs=[
            pl.BlockSpec(memory_space=pl.ANY),
            pl.BlockSpec(memory_space=pltpu.VMEM),
            pl.BlockSpec(memory_space=pltpu.VMEM),
        ],
        out_specs=pl.BlockSpec(memory_space=pl.ANY),
        out_shape=jax.ShapeDtypeStruct((n, out_f), jnp.float32),
        scratch_shapes=[
            pltpu.VMEM((_NBUF, _CHUNK, in_f), jnp.float32),
            pltpu.VMEM((_NBUF, _CHUNK, out_f), jnp.float32),
            pltpu.SemaphoreType.DMA((_NBUF,)),
            pltpu.SemaphoreType.DMA((_NBUF,)),
        ],
    )(input, W, b2)
    return out


if __name__ == "__main__":
    import numpy as np

    x = np.random.randn(65536, 256).astype(np.float32)
    x *= (np.random.rand(65536, 256) < 0.01)
    W = np.random.randn(256, 64).astype(np.float32)
    b = np.random.randn(64).astype(np.float32)
    got = np.asarray(kernel(jnp.asarray(x), jnp.asarray(W), jnp.asarray(b)))
    want = x @ W + b
    print("max abs err:", np.abs(got - want).max())


# speedup vs baseline: 2.3326x; 2.3326x over previous
import jax
import jax.numpy as jnp
from jax import lax
from jax.experimental import pallas as pl
from jax.experimental.pallas import tpu as pltpu

_CHUNK = 2048
_NBUF = 8


def _body(x_hbm, wt_ref, b_ref, o_hbm, x_buf, o_buf, in_sems, out_sems):
    n = x_hbm.shape[0]
    num_chunks = n // _CHUNK
    wt = wt_ref[...]
    b_col = b_ref[...]

    def start_in(c, slot):
        pltpu.make_async_copy(
            x_hbm.at[pl.ds(c * _CHUNK, _CHUNK), :],
            x_buf.at[slot],
            in_sems.at[slot],
        ).start()

    for s in range(_NBUF):
        start_in(s, s)

    def step(c, _):
        slot = jax.lax.rem(c, _NBUF)
        pltpu.make_async_copy(
            x_hbm.at[pl.ds(c * _CHUNK, _CHUNK), :],
            x_buf.at[slot],
            in_sems.at[slot],
        ).wait()

        @pl.when(c >= _NBUF)
        def _():
            pltpu.make_async_copy(
                o_buf.at[slot],
                o_hbm.at[:, pl.ds((c - _NBUF) * _CHUNK, _CHUNK)],
                out_sems.at[slot],
            ).wait()

        o_buf[slot] = (
            lax.dot_general(
                wt,
                x_buf[slot],
                dimension_numbers=(((1,), (1,)), ((), ())),
                preferred_element_type=jnp.float32,
            )
            + b_col
        )
        pltpu.make_async_copy(
            o_buf.at[slot],
            o_hbm.at[:, pl.ds(c * _CHUNK, _CHUNK)],
            out_sems.at[slot],
        ).start()

        @pl.when(c + _NBUF < num_chunks)
        def _():
            start_in(c + _NBUF, slot)

        return _

    jax.lax.fori_loop(0, num_chunks, step, None)

    for s in range(_NBUF):
        c = num_chunks - _NBUF + s
        slot = jax.lax.rem(jnp.int32(c), _NBUF)
        pltpu.make_async_copy(
            o_buf.at[slot],
            o_hbm.at[:, pl.ds(c * _CHUNK, _CHUNK)],
            out_sems.at[slot],
        ).wait()


def kernel(input, W, b):
    n, in_f = input.shape
    out_f = W.shape[1]
    wt = W.T
    b_col = b.reshape(out_f, 1)
    out_t = pl.pallas_call(
        _body,
        in_specs=[
            pl.BlockSpec(memory_space=pl.ANY),
            pl.BlockSpec(memory_space=pltpu.VMEM),
            pl.BlockSpec(memory_space=pltpu.VMEM),
        ],
        out_specs=pl.BlockSpec(memory_space=pl.ANY),
        out_shape=jax.ShapeDtypeStruct((out_f, n), jnp.float32),
        scratch_shapes=[
            pltpu.VMEM((_NBUF, _CHUNK, in_f), jnp.float32),
            pltpu.VMEM((_NBUF, out_f, _CHUNK), jnp.float32),
            pltpu.SemaphoreType.DMA((_NBUF,)),
            pltpu.SemaphoreType.DMA((_NBUF,)),
        ],
    )(input, wt, b_col)
    return out_t.T


# speedup vs baseline: 2.4476x; 1.0493x over previous
import jax
import jax.numpy as jnp
from jax import lax
from jax.experimental import pallas as pl
from jax.experimental.pallas import tpu as pltpu

_CHUNK = 2048
_NBUF = 8


def _body(x_hbm, wt_ref, b_ref, o_hbm, x_buf, o_buf, in_sems, out_sems):
    n = x_hbm.shape[0]
    num_chunks = n // _CHUNK
    wt = wt_ref[...]
    b_col = jnp.transpose(b_ref[...])

    def start_in(c, slot):
        pltpu.make_async_copy(
            x_hbm.at[pl.ds(c * _CHUNK, _CHUNK), :],
            x_buf.at[slot],
            in_sems.at[slot],
        ).start()

    for s in range(_NBUF):
        start_in(s, s)

    def step(c, _):
        slot = jax.lax.rem(c, _NBUF)
        pltpu.make_async_copy(
            x_hbm.at[pl.ds(c * _CHUNK, _CHUNK), :],
            x_buf.at[slot],
            in_sems.at[slot],
        ).wait()

        @pl.when(c >= _NBUF)
        def _():
            pltpu.make_async_copy(
                o_buf.at[slot],
                o_hbm.at[:, pl.ds((c - _NBUF) * _CHUNK, _CHUNK)],
                out_sems.at[slot],
            ).wait()

        o_buf[slot] = (
            lax.dot_general(
                wt,
                x_buf[slot],
                dimension_numbers=(((1,), (1,)), ((), ())),
                preferred_element_type=jnp.float32,
            )
            + b_col
        )
        pltpu.make_async_copy(
            o_buf.at[slot],
            o_hbm.at[:, pl.ds(c * _CHUNK, _CHUNK)],
            out_sems.at[slot],
        ).start()

        @pl.when(c + _NBUF < num_chunks)
        def _():
            start_in(c + _NBUF, slot)

        return _

    jax.lax.fori_loop(0, num_chunks, step, None)

    for s in range(_NBUF):
        c = num_chunks - _NBUF + s
        slot = jax.lax.rem(jnp.int32(c), _NBUF)
        pltpu.make_async_copy(
            o_buf.at[slot],
            o_hbm.at[:, pl.ds(c * _CHUNK, _CHUNK)],
            out_sems.at[slot],
        ).wait()


def kernel(input, W, b):
    n, in_f = input.shape
    out_f = W.shape[1]
    wt = W.T
    b_row = b.reshape(1, out_f)
    out_t = pl.pallas_call(
        _body,
        in_specs=[
            pl.BlockSpec(memory_space=pl.ANY),
            pl.BlockSpec(memory_space=pltpu.VMEM),
            pl.BlockSpec(memory_space=pltpu.VMEM),
        ],
        out_specs=pl.BlockSpec(memory_space=pl.ANY),
        out_shape=jax.ShapeDtypeStruct((out_f, n), jnp.float32),
        scratch_shapes=[
            pltpu.VMEM((_NBUF, _CHUNK, in_f), jnp.float32),
            pltpu.VMEM((_NBUF, out_f, _CHUNK), jnp.float32),
            pltpu.SemaphoreType.DMA((_NBUF,)),
            pltpu.SemaphoreType.DMA((_NBUF,)),
        ],
    )(input, wt, b_row)
    return out_t.T
